# Initial kernel scaffold; baseline (speedup 1.0000x reference)
import jax
import jax.numpy as jnp
from jax.experimental import pallas as pl


def kernel(inputs, node_feature, edge_index, edge_type, W_rel1, W_self1, br1, W_rel2, W_self2, br2, ctx_table, Wm1, bm1, Wm2, bm2):
    raise NotImplementedError("write your pallas kernel here")


# speedup vs baseline: 2.2983x; 2.2983x over previous
import functools

import jax
import jax.numpy as jnp
from jax import lax
from jax.experimental import pallas as pl
from jax.experimental.pallas import tpu as pltpu
from jax.experimental.pallas import tpu_sc as plsc

N = 10000
E = 320000
R = 16
B = 4096

NC = 2
NS = 16
LN = 16
NPS = N // NS
K = 80

_MESH = plsc.VectorSubcoreMesh(
    core_axis_name="c", subcore_axis_name="s", num_cores=NC, num_subcores=NS)
_SC_PARAMS = pltpu.CompilerParams(use_tc_tiling_on_sc=False,
                                  needs_layout_passes=False)

_ZERO16 = functools.partial(jnp.zeros, (LN,), jnp.float32)


def _zero_accum(zbuf, accum, sid):
    def zb(i, _):
        for j in range(8):
            zbuf[i, pl.ds(j * LN, LN)] = _ZERO16()
        return _
    lax.fori_loop(0, 125, zb, None)
    for t in range(NPS // 125):
        pltpu.sync_copy(zbuf, accum.at[pl.ds(sid * NPS + t * 125, 125)])


def _cnt_body(dst_hbm, et_hbm, out0_hbm, out1_hbm, dv, ev, ix, ones, zbuf,
              accum):
    cid = lax.axis_index("c")
    sid = lax.axis_index("s")
    sps = (N * R) // NS

    def zb(i, _):
        zbuf[pl.ds(i * LN, LN)] = _ZERO16()
        return _
    lax.fori_loop(0, sps // LN, zb, None)
    pltpu.sync_copy(zbuf, accum.at[pl.ds(sid * sps, sps)])
    for z in range(K // LN):
        ones[pl.ds(z * LN, LN)] = jnp.ones((LN,), jnp.float32)
    plsc.subcore_barrier()

    epw = E // (NC * NS)
    wid = cid * NS + sid
    nblk = epw // K

    def blk(b, _):
        off = wid * epw + b * K
        pltpu.sync_copy(dst_hbm.at[pl.ds(off, K)], dv)
        pltpu.sync_copy(et_hbm.at[pl.ds(off, K)], ev)
        for z in range(K // LN):
            sl = pl.ds(z * LN, LN)
            ix[sl] = dv[sl] * R + ev[sl]
        pltpu.sync_copy(ones, accum.at[ix], add=True)
        return _
    lax.fori_loop(0, nblk, blk, None)

    plsc.subcore_barrier()

    @pl.when(cid == 0)
    def _():
        pltpu.sync_copy(accum.at[pl.ds(sid * sps, sps)],
                        out0_hbm.at[pl.ds(sid * sps, sps)])

    @pl.when(cid == 1)
    def _():
        pltpu.sync_copy(accum.at[pl.ds(sid * sps, sps)],
                        out1_hbm.at[pl.ds(sid * sps, sps)])


def _count_edges(dst, et):
    k = pl.kernel(
        _cnt_body,
        out_type=(jax.ShapeDtypeStruct((N * R,), jnp.float32),
                  jax.ShapeDtypeStruct((N * R,), jnp.float32)),
        mesh=_MESH,
        compiler_params=_SC_PARAMS,
        scratch_types=[
            pltpu.VMEM((K,), jnp.int32),
            pltpu.VMEM((K,), jnp.int32),
            pltpu.VMEM((K,), jnp.int32),
            pltpu.VMEM((K,), jnp.float32),
            pltpu.VMEM(((N * R) // NS,), jnp.float32),
            pltpu.VMEM_SHARED((N * R,), jnp.float32),
        ],
    )
    return k(dst, et)


def _inv_kernel(c0_ref, c1_ref, o_ref):
    o_ref[...] = 1.0 / jnp.maximum(c0_ref[...] + c1_ref[...], 1.0)


def _inv_flat(cnt0, cnt1):
    shp = jax.ShapeDtypeStruct((N * R // 128, 128), jnp.float32)
    return pl.pallas_call(_inv_kernel, out_shape=shp)(
        cnt0.reshape(N * R // 128, 128), cnt1.reshape(N * R // 128, 128))


def _agg_body(y_hbm, src_hbm, dst_hbm, et_hbm, inv_hbm, out_hbm,
              sv, dv, ev, gx, ix, iscl, rows, zbuf, accum, sem1, sem2):
    cid = lax.axis_index("c")
    sid = lax.axis_index("s")
    _zero_accum(zbuf, accum, sid)
    plsc.subcore_barrier()

    eps = E // NS
    nblk = eps // K

    def blk(b, _):
        off = sid * eps + b * K
        pltpu.sync_copy(src_hbm.at[pl.ds(off, K)], sv)
        pltpu.sync_copy(dst_hbm.at[pl.ds(off, K)], dv)
        pltpu.sync_copy(et_hbm.at[pl.ds(off, K)], ev)
        for z in range(K // LN):
            sl = pl.ds(z * LN, LN)
            s16 = sv[sl]
            e16 = ev[sl]
            d16 = dv[sl]
            gx[sl] = s16 * (2 * R) + e16 * 2 + cid
            ix[sl] = d16 * R + e16
        cp1 = pltpu.async_copy(y_hbm.at[gx], rows, sem1)
        cp2 = pltpu.async_copy(inv_hbm.at[ix], iscl, sem2)
        cp1.wait()
        cp2.wait()

        def scale(e, _):
            sp = plsc.load_gather(iscl, [jnp.broadcast_to(e, (LN,))])
            for j in range(8):
                sl = pl.ds(j * LN, LN)
                rows[e, sl] = rows[e, sl] * sp
            return _
        lax.fori_loop(0, K, scale, None)
        pltpu.sync_copy(rows, accum.at[dv], add=True)
        return _
    lax.fori_loop(0, nblk, blk, None)

    plsc.subcore_barrier()
    pltpu.sync_copy(accum.at[pl.ds(sid * NPS, NPS)],
                    out_hbm.at[cid, pl.ds(sid * NPS, NPS), :])


def _aggregate(y2d, src, dst, et, invf):
    k = pl.kernel(
        _agg_body,
        out_type=jax.ShapeDtypeStruct((NC, N, 128), jnp.float32),
        mesh=_MESH,
        compiler_params=_SC_PARAMS,
        scratch_types=[
            pltpu.VMEM((K,), jnp.int32),
            pltpu.VMEM((K,), jnp.int32),
            pltpu.VMEM((K,), jnp.int32),
            pltpu.VMEM((K,), jnp.int32),
            pltpu.VMEM((K,), jnp.int32),
            pltpu.VMEM((K,), jnp.float32),
            pltpu.VMEM((K, 128), jnp.float32),
            pltpu.VMEM((125, 128), jnp.float32),
            pltpu.VMEM_SHARED((N, 128), jnp.float32),
            pltpu.SemaphoreType.DMA,
            pltpu.SemaphoreType.DMA,
        ],
    )
    return k(y2d, src, dst, et, invf)


def _mm_kernel(x_ref, w_ref, o_ref):
    o_ref[...] = jnp.dot(x_ref[...], w_ref[...],
                         preferred_element_type=jnp.float32)


def _matmul(x, w):
    n, d = x.shape
    m = w.shape[1]
    BN = 1000
    return pl.pallas_call(
        _mm_kernel,
        grid=(n // BN,),
        in_specs=[pl.BlockSpec((BN, d), lambda i: (i, 0)),
                  pl.BlockSpec((d, m), lambda i: (0, 0))],
        out_specs=pl.BlockSpec((BN, m), lambda i: (i, 0)),
        out_shape=jax.ShapeDtypeStruct((n, m), jnp.float32),
    )(x, w)


def _fin_kernel(a_ref, x_ref, w_ref, b_ref, o_ref, *, res):
    a = jnp.concatenate([a_ref[0], a_ref[1]], axis=-1)
    h = jax.nn.relu(a + jnp.dot(x_ref[...], w_ref[...],
                                preferred_element_type=jnp.float32) + b_ref[...])
    if res:
        h = h + x_ref[...]
    o_ref[...] = h


def _finish(a2, x, w_self, bias, res):
    d = x.shape[1]
    BN = 1000
    return pl.pallas_call(
        functools.partial(_fin_kernel, res=res),
        grid=(N // BN,),
        in_specs=[pl.BlockSpec((NC, BN, 128), lambda i: (0, i, 0)),
                  pl.BlockSpec((BN, d), lambda i: (i, 0)),
                  pl.BlockSpec((d, 256), lambda i: (0, 0)),
                  pl.BlockSpec((1, 256), lambda i: (0, 0))],
        out_specs=pl.BlockSpec((BN, 256), lambda i: (i, 0)),
        out_shape=jax.ShapeDtypeStruct((N, 256), jnp.float32),
    )(a2, x, w_self, bias)


def _gath_body(emb_hbm, ctab_hbm, i1_hbm, i2_hbm, i3_hbm,
               d1a_hbm, d1b_hbm, d2a_hbm, d2b_hbm, cc_hbm,
               iv, gx, r1, sem1):
    cid = lax.axis_index("c")
    sid = lax.axis_index("s")
    wid = cid * NS + sid
    q = B // (NC * NS)
    base = wid * q

    def emb_gather(idx_hbm, half, out_hbm):
        pltpu.sync_copy(idx_hbm.at[pl.ds(base, q)], iv)
        for z in range(q // LN):
            sl = pl.ds(z * LN, LN)
            gx[sl] = iv[sl] * 2 + half
        pltpu.async_copy(emb_hbm.at[gx], r1, sem1).wait()
        pltpu.sync_copy(r1, out_hbm.at[pl.ds(base, q)])

    emb_gather(i1_hbm, 0, d1a_hbm)
    emb_gather(i1_hbm, 1, d1b_hbm)
    emb_gather(i2_hbm, 0, d2a_hbm)
    emb_gather(i2_hbm, 1, d2b_hbm)
    pltpu.sync_copy(i3_hbm.at[pl.ds(base, q)], iv)
    pltpu.async_copy(ctab_hbm.at[iv], r1, sem1).wait()
    pltpu.sync_copy(r1, cc_hbm.at[pl.ds(base, q)])


def _head_gather(emb2, ctab, i1, i2, i3):
    q = B // (NC * NS)
    out = jax.ShapeDtypeStruct((B, 128), jnp.float32)
    k = pl.kernel(
        _gath_body,
        out_type=(out,) * 5,
        mesh=_MESH,
        compiler_params=_SC_PARAMS,
        scratch_types=[
            pltpu.VMEM((q,), jnp.int32),
            pltpu.VMEM((q,), jnp.int32),
            pltpu.VMEM((q, 128), jnp.float32),
            pltpu.SemaphoreType.DMA,
        ],
    )
    return k(emb2, ctab, i1, i2, i3)


def _head_kernel(d1a_ref, d1b_ref, d2a_ref, d2b_ref, cc_ref,
                 wa1_ref, wa2_ref, wb1_ref, wb2_ref, wc_ref, b1_ref,
                 w2_ref, b2_ref, o_ref):
    dot = functools.partial(jnp.dot, preferred_element_type=jnp.float32)
    hid = jax.nn.relu(
        dot(d1a_ref[...], wa1_ref[...]) + dot(d1b_ref[...], wa2_ref[...])
        + dot(d2a_ref[...], wb1_ref[...]) + dot(d2b_ref[...], wb2_ref[...])
        + dot(cc_ref[...], wc_ref[...]) + b1_ref[...])
    o_ref[...] = jnp.sum(hid * w2_ref[...], axis=1, keepdims=True) + b2_ref[...]


def _head(d1a, d1b, d2a, d2b, cc, wm1, bm1, wm2, bm2):
    wc = jnp.pad(wm1[512:], ((0, 64), (0, 0)))
    return pl.pallas_call(
        _head_kernel,
        out_shape=jax.ShapeDtypeStruct((B, 1), jnp.float32),
    )(d1a, d1b, d2a, d2b, cc, wm1[:128], wm1[128:256], wm1[256:384],
      wm1[384:512], wc, bm1.reshape(1, 512), wm2.reshape(1, 512),
      bm2.reshape(1, 1))


def kernel(inputs, node_feature, edge_index, edge_type, W_rel1, W_self1, br1,
           W_rel2, W_self2, br2, ctx_table, Wm1, bm1, Wm2, bm2):
    src = edge_index[0]
    dst = edge_index[1]
    et = edge_type

    cnt0, cnt1 = _count_edges(dst, et)
    invf = _inv_flat(cnt0, cnt1).reshape(N * R)

    Wt1 = W_rel1.reshape(R, 128, 256).transpose(1, 0, 2).reshape(128, R * 256)
    y1 = _matmul(node_feature, Wt1)
    a1 = _aggregate(y1.reshape(N * R * 2, 128), src, dst, et, invf)
    h1 = _finish(a1, node_feature, W_self1, br1.reshape(1, 256), res=False)

    Wt2 = W_rel2.reshape(R, 256, 256).transpose(1, 0, 2).reshape(256, R * 256)
    y2 = _matmul(h1, Wt2)
    a2 = _aggregate(y2.reshape(N * R * 2, 128), src, dst, et, invf)
    h2 = _finish(a2, h1, W_self2, br2.reshape(1, 256), res=True)

    ctab = jnp.pad(ctx_table, ((0, 0), (0, 64)))
    d1a, d1b, d2a, d2b, cc = _head_gather(
        h2.reshape(2 * N, 128), ctab,
        inputs[:, 0], inputs[:, 1], inputs[:, 2])
    out = _head(d1a, d1b, d2a, d2b, cc, Wm1, bm1, Wm2, bm2)
    return out.reshape(B)


# speedup vs baseline: 3.7596x; 1.6358x over previous
import functools

import jax
import jax.numpy as jnp
from jax import lax
from jax.experimental import pallas as pl
from jax.experimental.pallas import tpu as pltpu
from jax.experimental.pallas import tpu_sc as plsc

N = 10000
E = 320000
R = 16
B = 4096

NC = 2
NS = 16
LN = 16
NPS = N // NS
K = 80

_MESH = plsc.VectorSubcoreMesh(
    core_axis_name="c", subcore_axis_name="s", num_cores=NC, num_subcores=NS)
_SC_PARAMS = pltpu.CompilerParams(use_tc_tiling_on_sc=False,
                                  needs_layout_passes=False)

_ZERO16 = functools.partial(jnp.zeros, (LN,), jnp.float32)


def _zero_accum(zbuf, accum, sid):
    def zb(i, _):
        for j in range(8):
            zbuf[i, pl.ds(j * LN, LN)] = _ZERO16()
        return _
    lax.fori_loop(0, K, zb, None)
    for t in range(NPS // K):
        pltpu.sync_copy(zbuf, accum.at[pl.ds(sid * NPS + t * K, K)])
    rem = NPS % K
    if rem:
        pltpu.sync_copy(zbuf.at[pl.ds(0, rem)],
                        accum.at[pl.ds(sid * NPS + (NPS // K) * K, rem)])


def _cnt_body(dst_hbm, et_hbm, out0_hbm, out1_hbm, dv, ev, ix, ones, zbuf,
              accum):
    cid = lax.axis_index("c")
    sid = lax.axis_index("s")
    sps = (N * R) // NS

    def zb(i, _):
        zbuf[pl.ds(i * LN, LN)] = _ZERO16()
        return _
    lax.fori_loop(0, sps // LN, zb, None)
    pltpu.sync_copy(zbuf, accum.at[pl.ds(sid * sps, sps)])
    for z in range(K // LN):
        ones[pl.ds(z * LN, LN)] = jnp.ones((LN,), jnp.float32)
    plsc.subcore_barrier()

    epw = E // (NC * NS)
    wid = cid * NS + sid
    nblk = epw // K

    def blk(b, _):
        off = wid * epw + b * K
        pltpu.sync_copy(dst_hbm.at[pl.ds(off, K)], dv)
        pltpu.sync_copy(et_hbm.at[pl.ds(off, K)], ev)
        for z in range(K // LN):
            sl = pl.ds(z * LN, LN)
            ix[sl] = dv[sl] * R + ev[sl]
        pltpu.sync_copy(ones, accum.at[ix], add=True)
        return _
    lax.fori_loop(0, nblk, blk, None)

    plsc.subcore_barrier()

    @pl.when(cid == 0)
    def _():
        pltpu.sync_copy(accum.at[pl.ds(sid * sps, sps)],
                        out0_hbm.at[pl.ds(sid * sps, sps)])

    @pl.when(cid == 1)
    def _():
        pltpu.sync_copy(accum.at[pl.ds(sid * sps, sps)],
                        out1_hbm.at[pl.ds(sid * sps, sps)])


def _count_edges(dst, et):
    k = pl.kernel(
        _cnt_body,
        out_type=(jax.ShapeDtypeStruct((N * R,), jnp.float32),
                  jax.ShapeDtypeStruct((N * R,), jnp.float32)),
        mesh=_MESH,
        compiler_params=_SC_PARAMS,
        scratch_types=[
            pltpu.VMEM((K,), jnp.int32),
            pltpu.VMEM((K,), jnp.int32),
            pltpu.VMEM((K,), jnp.int32),
            pltpu.VMEM((K,), jnp.float32),
            pltpu.VMEM(((N * R) // NS,), jnp.float32),
            pltpu.VMEM_SHARED((N * R,), jnp.float32),
        ],
    )
    return k(dst, et)


def _inv_kernel(c0_ref, c1_ref, o_ref):
    o_ref[...] = 1.0 / jnp.maximum(c0_ref[...] + c1_ref[...], 1.0)


def _inv_flat(cnt0, cnt1):
    shp = jax.ShapeDtypeStruct((N * R // 128, 128), jnp.float32)
    return pl.pallas_call(_inv_kernel, out_shape=shp)(
        cnt0.reshape(N * R // 128, 128), cnt1.reshape(N * R // 128, 128))


EPS = E // NS
NBLK = EPS // K


def _agg_body(y_hbm, src_hbm, dst_hbm, et_hbm, inv_hbm, out_hbm,
              sv0, sv1, dv0, dv1, ev0, ev1, gx0, gx1, ix0, ix1,
              iscl0, iscl1, rows0, rows1, accum,
              semx0, semx1, semr0, semr1, semi0, semi1):
    cid = lax.axis_index("c")
    sid = lax.axis_index("s")
    bufs = ((sv0, dv0, ev0, gx0, ix0, iscl0, rows0, semx0, semr0, semi0),
            (sv1, dv1, ev1, gx1, ix1, iscl1, rows1, semx1, semr1, semi1))
    base = sid * EPS
    _zero_accum(rows0, accum, sid)
    plsc.subcore_barrier()

    def idx_copies(b, buf):
        svb, dvb, evb, _gx, _ix, _is, _ro, semx, _sr, _si = buf
        off = base + b * K
        return (pltpu.make_async_copy(src_hbm.at[pl.ds(off, K)], svb, semx),
                pltpu.make_async_copy(dst_hbm.at[pl.ds(off, K)], dvb, semx),
                pltpu.make_async_copy(et_hbm.at[pl.ds(off, K)], evb, semx))

    def fire_idx(b, buf):
        for cp in idx_copies(b, buf):
            cp.start()

    def fire_gather(b, buf):
        svb, dvb, evb, gxb, ixb, isclb, rowsb, semx, semr, semi = buf
        for cp in idx_copies(b, buf):
            cp.wait()
        for z in range(K // LN):
            sl = pl.ds(z * LN, LN)
            s16 = svb[sl]
            e16 = evb[sl]
            d16 = dvb[sl]
            gxb[sl] = s16 * (2 * R) + e16 * 2 + cid
            ixb[sl] = d16 * R + e16
        pltpu.make_async_copy(y_hbm.at[gxb], rowsb, semr).start()
        pltpu.make_async_copy(inv_hbm.at[ixb], isclb, semi).start()

    def finish(b, buf):
        svb, dvb, evb, gxb, ixb, isclb, rowsb, semx, semr, semi = buf
        pltpu.make_async_copy(y_hbm.at[gxb], rowsb, semr).wait()
        pltpu.make_async_copy(inv_hbm.at[ixb], isclb, semi).wait()

        def scale(e, _):
            sp = plsc.load_gather(isclb, [jnp.broadcast_to(e, (LN,))])
            for j in range(8):
                sl = pl.ds(j * LN, LN)
                rowsb[e, sl] = rowsb[e, sl] * sp
            return _
        lax.fori_loop(0, K, scale, None)
        pltpu.sync_copy(rowsb, accum.at[dvb], add=True)

    fire_idx(0, bufs[0])
    fire_idx(1, bufs[1])
    fire_gather(0, bufs[0])

    def pair(g, _):
        b = 2 * g
        for i in range(2):
            bi = b + i
            nxt = bufs[(i + 1) % 2]

            @pl.when(bi + 1 < NBLK)
            def _():
                fire_gather(bi + 1, nxt)
            finish(bi, bufs[i])

            @pl.when(bi + 2 < NBLK)
            def _():
                fire_idx(bi + 2, bufs[i])
        return _
    lax.fori_loop(0, NBLK // 2, pair, None)

    plsc.subcore_barrier()
    pltpu.sync_copy(accum.at[pl.ds(sid * NPS, NPS)],
                    out_hbm.at[cid, pl.ds(sid * NPS, NPS), :])


def _aggregate(y2d, src, dst, et, invf):
    k = pl.kernel(
        _agg_body,
        out_type=jax.ShapeDtypeStruct((NC, N, 128), jnp.float32),
        mesh=_MESH,
        compiler_params=_SC_PARAMS,
        scratch_types=(
            [pltpu.VMEM((K,), jnp.int32)] * 10
            + [pltpu.VMEM((K,), jnp.float32)] * 2
            + [pltpu.VMEM((K, 128), jnp.float32)] * 2
            + [pltpu.VMEM_SHARED((N, 128), jnp.float32)]
            + [pltpu.SemaphoreType.DMA] * 6
        ),
    )
    return k(y2d, src, dst, et, invf)


def _mm_kernel(x_ref, w_ref, o_ref):
    o_ref[...] = jnp.dot(x_ref[...], w_ref[...],
                         preferred_element_type=jnp.float32)


def _matmul(x, w):
    n, d = x.shape
    m = w.shape[1]
    BN = 1000
    return pl.pallas_call(
        _mm_kernel,
        grid=(n // BN,),
        in_specs=[pl.BlockSpec((BN, d), lambda i: (i, 0)),
                  pl.BlockSpec((d, m), lambda i: (0, 0))],
        out_specs=pl.BlockSpec((BN, m), lambda i: (i, 0)),
        out_shape=jax.ShapeDtypeStruct((n, m), jnp.float32),
    )(x, w)


def _fin_kernel(a_ref, x_ref, w_ref, b_ref, o_ref, *, res):
    a = jnp.concatenate([a_ref[0], a_ref[1]], axis=-1)
    h = jax.nn.relu(a + jnp.dot(x_ref[...], w_ref[...],
                                preferred_element_type=jnp.float32) + b_ref[...])
    if res:
        h = h + x_ref[...]
    o_ref[...] = h


def _finish(a2, x, w_self, bias, res):
    d = x.shape[1]
    BN = 1000
    return pl.pallas_call(
        functools.partial(_fin_kernel, res=res),
        grid=(N // BN,),
        in_specs=[pl.BlockSpec((NC, BN, 128), lambda i: (0, i, 0)),
                  pl.BlockSpec((BN, d), lambda i: (i, 0)),
                  pl.BlockSpec((d, 256), lambda i: (0, 0)),
                  pl.BlockSpec((1, 256), lambda i: (0, 0))],
        out_specs=pl.BlockSpec((BN, 256), lambda i: (i, 0)),
        out_shape=jax.ShapeDtypeStruct((N, 256), jnp.float32),
    )(a2, x, w_self, bias)


def _gath_body(emb_hbm, ctab_hbm, i1_hbm, i2_hbm, i3_hbm,
               d1a_hbm, d1b_hbm, d2a_hbm, d2b_hbm, cc_hbm,
               iv, gx, r1, sem1):
    cid = lax.axis_index("c")
    sid = lax.axis_index("s")
    wid = cid * NS + sid
    q = B // (NC * NS)
    base = wid * q

    def emb_gather(idx_hbm, half, out_hbm):
        pltpu.sync_copy(idx_hbm.at[pl.ds(base, q)], iv)
        for z in range(q // LN):
            sl = pl.ds(z * LN, LN)
            gx[sl] = iv[sl] * 2 + half
        pltpu.async_copy(emb_hbm.at[gx], r1, sem1).wait()
        pltpu.sync_copy(r1, out_hbm.at[pl.ds(base, q)])

    emb_gather(i1_hbm, 0, d1a_hbm)
    emb_gather(i1_hbm, 1, d1b_hbm)
    emb_gather(i2_hbm, 0, d2a_hbm)
    emb_gather(i2_hbm, 1, d2b_hbm)
    pltpu.sync_copy(i3_hbm.at[pl.ds(base, q)], iv)
    pltpu.async_copy(ctab_hbm.at[iv], r1, sem1).wait()
    pltpu.sync_copy(r1, cc_hbm.at[pl.ds(base, q)])


def _head_gather(emb2, ctab, i1, i2, i3):
    q = B // (NC * NS)
    out = jax.ShapeDtypeStruct((B, 128), jnp.float32)
    k = pl.kernel(
        _gath_body,
        out_type=(out,) * 5,
        mesh=_MESH,
        compiler_params=_SC_PARAMS,
        scratch_types=[
            pltpu.VMEM((q,), jnp.int32),
            pltpu.VMEM((q,), jnp.int32),
            pltpu.VMEM((q, 128), jnp.float32),
            pltpu.SemaphoreType.DMA,
        ],
    )
    return k(emb2, ctab, i1, i2, i3)


def _head_kernel(d1a_ref, d1b_ref, d2a_ref, d2b_ref, cc_ref,
                 wa1_ref, wa2_ref, wb1_ref, wb2_ref, wc_ref, b1_ref,
                 w2_ref, b2_ref, o_ref):
    dot = functools.partial(jnp.dot, preferred_element_type=jnp.float32)
    hid = jax.nn.relu(
        dot(d1a_ref[...], wa1_ref[...]) + dot(d1b_ref[...], wa2_ref[...])
        + dot(d2a_ref[...], wb1_ref[...]) + dot(d2b_ref[...], wb2_ref[...])
        + dot(cc_ref[...], wc_ref[...]) + b1_ref[...])
    o_ref[...] = jnp.sum(hid * w2_ref[...], axis=1, keepdims=True) + b2_ref[...]


def _head(d1a, d1b, d2a, d2b, cc, wm1, bm1, wm2, bm2):
    wc = jnp.pad(wm1[512:], ((0, 64), (0, 0)))
    return pl.pallas_call(
        _head_kernel,
        out_shape=jax.ShapeDtypeStruct((B, 1), jnp.float32),
    )(d1a, d1b, d2a, d2b, cc, wm1[:128], wm1[128:256], wm1[256:384],
      wm1[384:512], wc, bm1.reshape(1, 512), wm2.reshape(1, 512),
      bm2.reshape(1, 1))


def kernel(inputs, node_feature, edge_index, edge_type, W_rel1, W_self1, br1,
           W_rel2, W_self2, br2, ctx_table, Wm1, bm1, Wm2, bm2):
    src = edge_index[0]
    dst = edge_index[1]
    et = edge_type

    cnt0, cnt1 = _count_edges(dst, et)
    invf = _inv_flat(cnt0, cnt1).reshape(N * R)

    Wt1 = W_rel1.reshape(R, 128, 256).transpose(1, 0, 2).reshape(128, R * 256)
    y1 = _matmul(node_feature, Wt1)
    a1 = _aggregate(y1.reshape(N * R * 2, 128), src, dst, et, invf)
    h1 = _finish(a1, node_feature, W_self1, br1.reshape(1, 256), res=False)

    Wt2 = W_rel2.reshape(R, 256, 256).transpose(1, 0, 2).reshape(256, R * 256)
    y2 = _matmul(h1, Wt2)
    a2 = _aggregate(y2.reshape(N * R * 2, 128), src, dst, et, invf)
    h2 = _finish(a2, h1, W_self2, br2.reshape(1, 256), res=True)

    ctab = jnp.pad(ctx_table, ((0, 0), (0, 64)))
    d1a, d1b, d2a, d2b, cc = _head_gather(
        h2.reshape(2 * N, 128), ctab,
        inputs[:, 0], inputs[:, 1], inputs[:, 2])
    out = _head(d1a, d1b, d2a, d2b, cc, Wm1, bm1, Wm2, bm2)
    return out.reshape(B)


# speedup vs baseline: 4.5996x; 1.2234x over previous
import functools

import jax
import jax.numpy as jnp
from jax import lax
from jax.experimental import pallas as pl
from jax.experimental.pallas import tpu as pltpu
from jax.experimental.pallas import tpu_sc as plsc

N = 10000
E = 320000
R = 16
B = 4096

NC = 2
NS = 16
LN = 16
NPS = N // NS
K = 80

_MESH = plsc.VectorSubcoreMesh(
    core_axis_name="c", subcore_axis_name="s", num_cores=NC, num_subcores=NS)
_SC_PARAMS = pltpu.CompilerParams(use_tc_tiling_on_sc=False,
                                  needs_layout_passes=False)

_ZERO16 = functools.partial(jnp.zeros, (LN,), jnp.float32)


def _zero_accum(zbuf, accum, sid):
    def zb(i, _):
        for j in range(8):
            zbuf[i, pl.ds(j * LN, LN)] = _ZERO16()
        return _
    lax.fori_loop(0, K, zb, None)
    for t in range(NPS // K):
        pltpu.sync_copy(zbuf, accum.at[pl.ds(sid * NPS + t * K, K)])
    rem = NPS % K
    if rem:
        pltpu.sync_copy(zbuf.at[pl.ds(0, rem)],
                        accum.at[pl.ds(sid * NPS + (NPS // K) * K, rem)])


def _cnt_body(dst_hbm, et_hbm, out0_hbm, out1_hbm, dv, ev, ix, ones, zbuf,
              accum):
    cid = lax.axis_index("c")
    sid = lax.axis_index("s")
    sps = (N * R) // NS

    def zb(i, _):
        zbuf[pl.ds(i * LN, LN)] = _ZERO16()
        return _
    lax.fori_loop(0, sps // LN, zb, None)
    pltpu.sync_copy(zbuf, accum.at[pl.ds(sid * sps, sps)])
    for z in range(K // LN):
        ones[pl.ds(z * LN, LN)] = jnp.ones((LN,), jnp.float32)
    plsc.subcore_barrier()

    epw = E // (NC * NS)
    wid = cid * NS + sid
    nblk = epw // K

    def blk(b, _):
        off = wid * epw + b * K
        pltpu.sync_copy(dst_hbm.at[pl.ds(off, K)], dv)
        pltpu.sync_copy(et_hbm.at[pl.ds(off, K)], ev)
        for z in range(K // LN):
            sl = pl.ds(z * LN, LN)
            ix[sl] = dv[sl] * R + ev[sl]
        pltpu.sync_copy(ones, accum.at[ix], add=True)
        return _
    lax.fori_loop(0, nblk, blk, None)

    plsc.subcore_barrier()

    @pl.when(cid == 0)
    def _():
        pltpu.sync_copy(accum.at[pl.ds(sid * sps, sps)],
                        out0_hbm.at[pl.ds(sid * sps, sps)])

    @pl.when(cid == 1)
    def _():
        pltpu.sync_copy(accum.at[pl.ds(sid * sps, sps)],
                        out1_hbm.at[pl.ds(sid * sps, sps)])


def _count_edges(dst, et):
    k = pl.kernel(
        _cnt_body,
        out_type=(jax.ShapeDtypeStruct((N * R,), jnp.float32),
                  jax.ShapeDtypeStruct((N * R,), jnp.float32)),
        mesh=_MESH,
        compiler_params=_SC_PARAMS,
        scratch_types=[
            pltpu.VMEM((K,), jnp.int32),
            pltpu.VMEM((K,), jnp.int32),
            pltpu.VMEM((K,), jnp.int32),
            pltpu.VMEM((K,), jnp.float32),
            pltpu.VMEM(((N * R) // NS,), jnp.float32),
            pltpu.VMEM_SHARED((N * R,), jnp.float32),
        ],
    )
    return k(dst, et)


def _inv_kernel(c0_ref, c1_ref, o_ref):
    o_ref[...] = 1.0 / jnp.maximum(c0_ref[...] + c1_ref[...], 1.0)


def _inv_flat(cnt0, cnt1):
    shp = jax.ShapeDtypeStruct((N * R // 128, 128), jnp.float32)
    return pl.pallas_call(_inv_kernel, out_shape=shp)(
        cnt0.reshape(N * R // 128, 128), cnt1.reshape(N * R // 128, 128))


EPS = E // NS
NBLK = EPS // K


def _agg_body(y_hbm, src_hbm, dst_hbm, et_hbm, inv_hbm, out_hbm,
              sv0, sv1, dv0, dv1, ev0, ev1, gx0, gx1, ix0, ix1,
              iscl0, iscl1, rows0, rows1, accum,
              semx0, semx1, semr0, semr1, semi0, semi1):
    cid = lax.axis_index("c")
    sid = lax.axis_index("s")
    bufs = ((sv0, dv0, ev0, gx0, ix0, iscl0, rows0, semx0, semr0, semi0),
            (sv1, dv1, ev1, gx1, ix1, iscl1, rows1, semx1, semr1, semi1))
    base = sid * EPS
    _zero_accum(rows0, accum, sid)
    plsc.subcore_barrier()

    def idx_copies(b, buf):
        svb, dvb, evb, _gx, _ix, _is, _ro, semx, _sr, _si = buf
        off = base + b * K
        return (pltpu.make_async_copy(src_hbm.at[pl.ds(off, K)], svb, semx),
                pltpu.make_async_copy(dst_hbm.at[pl.ds(off, K)], dvb, semx),
                pltpu.make_async_copy(et_hbm.at[pl.ds(off, K)], evb, semx))

    def fire_idx(b, buf):
        for cp in idx_copies(b, buf):
            cp.start()

    def fire_gather(b, buf):
        svb, dvb, evb, gxb, ixb, isclb, rowsb, semx, semr, semi = buf
        for cp in idx_copies(b, buf):
            cp.wait()
        for z in range(K // LN):
            sl = pl.ds(z * LN, LN)
            s16 = svb[sl]
            e16 = evb[sl]
            d16 = dvb[sl]
            gxb[sl] = (e16 * 2 + cid) * N + s16
            ixb[sl] = d16 * R + e16
        pltpu.make_async_copy(y_hbm.at[gxb], rowsb, semr).start()
        pltpu.make_async_copy(inv_hbm.at[ixb], isclb, semi).start()

    def finish(b, buf):
        svb, dvb, evb, gxb, ixb, isclb, rowsb, semx, semr, semi = buf
        pltpu.make_async_copy(y_hbm.at[gxb], rowsb, semr).wait()
        pltpu.make_async_copy(inv_hbm.at[ixb], isclb, semi).wait()

        def scale(g, _):
            base = g * LN
            for eo in range(LN):
                e = base + eo
                sp = plsc.load_gather(isclb, [jnp.broadcast_to(e, (LN,))])
                for j in range(8):
                    sl = pl.ds(j * LN, LN)
                    rowsb[e, sl] = rowsb[e, sl] * sp
            return _
        lax.fori_loop(0, K // LN, scale, None)
        pltpu.sync_copy(rowsb, accum.at[dvb], add=True)

    fire_idx(0, bufs[0])
    fire_idx(1, bufs[1])
    fire_gather(0, bufs[0])

    def pair(g, _):
        b = 2 * g
        for i in range(2):
            bi = b + i
            nxt = bufs[(i + 1) % 2]

            @pl.when(bi + 1 < NBLK)
            def _():
                fire_gather(bi + 1, nxt)
            finish(bi, bufs[i])

            @pl.when(bi + 2 < NBLK)
            def _():
                fire_idx(bi + 2, bufs[i])
        return _
    lax.fori_loop(0, NBLK // 2, pair, None)

    plsc.subcore_barrier()
    pltpu.sync_copy(accum.at[pl.ds(sid * NPS, NPS)],
                    out_hbm.at[cid, pl.ds(sid * NPS, NPS), :])


def _aggregate(y2d, src, dst, et, invf):
    k = pl.kernel(
        _agg_body,
        out_type=jax.ShapeDtypeStruct((NC, N, 128), jnp.float32),
        mesh=_MESH,
        compiler_params=_SC_PARAMS,
        scratch_types=(
            [pltpu.VMEM((K,), jnp.int32)] * 10
            + [pltpu.VMEM((K,), jnp.float32)] * 2
            + [pltpu.VMEM((K, 128), jnp.float32)] * 2
            + [pltpu.VMEM_SHARED((N, 128), jnp.float32)]
            + [pltpu.SemaphoreType.DMA] * 6
        ),
    )
    return k(y2d, src, dst, et, invf)


def _mm_kernel(x_ref, w_ref, o_ref):
    r = jnp.dot(x_ref[...], w_ref[...], preferred_element_type=jnp.float32)
    for c in range(o_ref.shape[0]):
        o_ref[c] = r[:, c * 128:(c + 1) * 128]


def _matmul(x, w):
    n, d = x.shape
    m = w.shape[1]
    g = m // 128
    BN = 1000
    return pl.pallas_call(
        _mm_kernel,
        grid=(n // BN,),
        in_specs=[pl.BlockSpec((BN, d), lambda i: (i, 0)),
                  pl.BlockSpec((d, m), lambda i: (0, 0))],
        out_specs=pl.BlockSpec((g, BN, 128), lambda i: (0, i, 0)),
        out_shape=jax.ShapeDtypeStruct((g, n, 128), jnp.float32),
    )(x, w)


def _fin_kernel(a_ref, x_ref, w_ref, b_ref, o_ref, *, res):
    a = jnp.concatenate([a_ref[0], a_ref[1]], axis=-1)
    h = jax.nn.relu(a + jnp.dot(x_ref[...], w_ref[...],
                                preferred_element_type=jnp.float32) + b_ref[...])
    if res:
        h = h + x_ref[...]
    o_ref[...] = h


def _finish(a2, x, w_self, bias, res):
    d = x.shape[1]
    BN = 1000
    return pl.pallas_call(
        functools.partial(_fin_kernel, res=res),
        grid=(N // BN,),
        in_specs=[pl.BlockSpec((NC, BN, 128), lambda i: (0, i, 0)),
                  pl.BlockSpec((BN, d), lambda i: (i, 0)),
                  pl.BlockSpec((d, 256), lambda i: (0, 0)),
                  pl.BlockSpec((1, 256), lambda i: (0, 0))],
        out_specs=pl.BlockSpec((BN, 256), lambda i: (i, 0)),
        out_shape=jax.ShapeDtypeStruct((N, 256), jnp.float32),
    )(a2, x, w_self, bias)


def _gath_body(emb_hbm, ctab_hbm, i1_hbm, i2_hbm, i3_hbm,
               d1a_hbm, d1b_hbm, d2a_hbm, d2b_hbm, cc_hbm,
               iv, gx, r1, sem1):
    cid = lax.axis_index("c")
    sid = lax.axis_index("s")
    wid = cid * NS + sid
    q = B // (NC * NS)
    base = wid * q

    def emb_gather(idx_hbm, half, out_hbm):
        pltpu.sync_copy(idx_hbm.at[pl.ds(base, q)], iv)
        for z in range(q // LN):
            sl = pl.ds(z * LN, LN)
            gx[sl] = iv[sl] * 2 + half
        pltpu.async_copy(emb_hbm.at[gx], r1, sem1).wait()
        pltpu.sync_copy(r1, out_hbm.at[pl.ds(base, q)])

    emb_gather(i1_hbm, 0, d1a_hbm)
    emb_gather(i1_hbm, 1, d1b_hbm)
    emb_gather(i2_hbm, 0, d2a_hbm)
    emb_gather(i2_hbm, 1, d2b_hbm)
    pltpu.sync_copy(i3_hbm.at[pl.ds(base, q)], iv)
    pltpu.async_copy(ctab_hbm.at[iv], r1, sem1).wait()
    pltpu.sync_copy(r1, cc_hbm.at[pl.ds(base, q)])


def _head_gather(emb2, ctab, i1, i2, i3):
    q = B // (NC * NS)
    out = jax.ShapeDtypeStruct((B, 128), jnp.float32)
    k = pl.kernel(
        _gath_body,
        out_type=(out,) * 5,
        mesh=_MESH,
        compiler_params=_SC_PARAMS,
        scratch_types=[
            pltpu.VMEM((q,), jnp.int32),
            pltpu.VMEM((q,), jnp.int32),
            pltpu.VMEM((q, 128), jnp.float32),
            pltpu.SemaphoreType.DMA,
        ],
    )
    return k(emb2, ctab, i1, i2, i3)


def _head_kernel(d1a_ref, d1b_ref, d2a_ref, d2b_ref, cc_ref,
                 wa1_ref, wa2_ref, wb1_ref, wb2_ref, wc_ref, b1_ref,
                 w2_ref, b2_ref, o_ref):
    dot = functools.partial(jnp.dot, preferred_element_type=jnp.float32)
    hid = jax.nn.relu(
        dot(d1a_ref[...], wa1_ref[...]) + dot(d1b_ref[...], wa2_ref[...])
        + dot(d2a_ref[...], wb1_ref[...]) + dot(d2b_ref[...], wb2_ref[...])
        + dot(cc_ref[...], wc_ref[...]) + b1_ref[...])
    o_ref[...] = jnp.sum(hid * w2_ref[...], axis=1, keepdims=True) + b2_ref[...]


def _head(d1a, d1b, d2a, d2b, cc, wm1, bm1, wm2, bm2):
    wc = jnp.pad(wm1[512:], ((0, 64), (0, 0)))
    return pl.pallas_call(
        _head_kernel,
        out_shape=jax.ShapeDtypeStruct((B, 1), jnp.float32),
    )(d1a, d1b, d2a, d2b, cc, wm1[:128], wm1[128:256], wm1[256:384],
      wm1[384:512], wc, bm1.reshape(1, 512), wm2.reshape(1, 512),
      bm2.reshape(1, 1))


def kernel(inputs, node_feature, edge_index, edge_type, W_rel1, W_self1, br1,
           W_rel2, W_self2, br2, ctx_table, Wm1, bm1, Wm2, bm2):
    src = edge_index[0]
    dst = edge_index[1]
    et = edge_type

    cnt0, cnt1 = _count_edges(dst, et)
    invf = _inv_flat(cnt0, cnt1).reshape(N * R)

    Wt1 = W_rel1.reshape(R, 128, 256).transpose(1, 0, 2).reshape(128, R * 256)
    y1 = _matmul(node_feature, Wt1)
    a1 = _aggregate(y1.reshape(N * R * 2, 128), src, dst, et, invf)
    h1 = _finish(a1, node_feature, W_self1, br1.reshape(1, 256), res=False)

    Wt2 = W_rel2.reshape(R, 256, 256).transpose(1, 0, 2).reshape(256, R * 256)
    y2 = _matmul(h1, Wt2)
    a2 = _aggregate(y2.reshape(N * R * 2, 128), src, dst, et, invf)
    h2 = _finish(a2, h1, W_self2, br2.reshape(1, 256), res=True)

    ctab = jnp.pad(ctx_table, ((0, 0), (0, 64)))
    d1a, d1b, d2a, d2b, cc = _head_gather(
        h2.reshape(2 * N, 128), ctab,
        inputs[:, 0], inputs[:, 1], inputs[:, 2])
    out = _head(d1a, d1b, d2a, d2b, cc, Wm1, bm1, Wm2, bm2)
    return out.reshape(B)


# speedup vs baseline: 6.1202x; 1.3306x over previous
import functools

import jax
import jax.numpy as jnp
from jax import lax
from jax.experimental import pallas as pl
from jax.experimental.pallas import tpu as pltpu
from jax.experimental.pallas import tpu_sc as plsc

N = 10000
E = 320000
R = 16
B = 4096

NC = 2
NS = 16
LN = 16
NPS = N // NS
K = 80

_MESH = plsc.VectorSubcoreMesh(
    core_axis_name="c", subcore_axis_name="s", num_cores=NC, num_subcores=NS)
_SC_PARAMS = pltpu.CompilerParams(use_tc_tiling_on_sc=False,
                                  needs_layout_passes=False)

_ZERO16 = functools.partial(jnp.zeros, (LN,), jnp.float32)


def _zero_accum(zbuf, accum, sid):
    def zb(i, _):
        for j in range(8):
            zbuf[i, pl.ds(j * LN, LN)] = _ZERO16()
        return _
    lax.fori_loop(0, K, zb, None)
    for t in range(NPS // K):
        pltpu.sync_copy(zbuf, accum.at[pl.ds(sid * NPS + t * K, K)])
    rem = NPS % K
    if rem:
        pltpu.sync_copy(zbuf.at[pl.ds(0, rem)],
                        accum.at[pl.ds(sid * NPS + (NPS // K) * K, rem)])


_NIX = 5


def _cnt_body(dst_hbm, et_hbm, out0_hbm, out1_hbm, dv, ev,
              ix0, ix1, ix2, ix3, ix4, ones, zbuf, accum,
              sem0, sem1, sem2, sem3, sem4):
    cid = lax.axis_index("c")
    sid = lax.axis_index("s")
    sps = (N * R) // NS
    ixs = (ix0, ix1, ix2, ix3, ix4)
    sems = (sem0, sem1, sem2, sem3, sem4)

    def zb(i, _):
        zbuf[pl.ds(i * LN, LN)] = _ZERO16()
        return _
    lax.fori_loop(0, sps // LN, zb, None)
    pltpu.sync_copy(zbuf, accum.at[pl.ds(sid * sps, sps)])
    for z in range(K // LN):
        ones[pl.ds(z * LN, LN)] = jnp.ones((LN,), jnp.float32)

    epw = E // (NC * NS)
    wid = cid * NS + sid
    nblk = epw // K
    pltpu.sync_copy(dst_hbm.at[pl.ds(wid * epw, epw)], dv)
    pltpu.sync_copy(et_hbm.at[pl.ds(wid * epw, epw)], ev)
    plsc.subcore_barrier()

    def fire(b, i):
        for z in range(K // LN):
            sl = pl.ds(z * LN, LN)
            dsl = pl.ds(b * K + z * LN, LN)
            ixs[i][sl] = dv[dsl] * R + ev[dsl]
        pltpu.make_async_copy(ones, accum.at[ixs[i]], sems[i]).start()

    def drain(i):
        pltpu.make_async_copy(ones, accum.at[ixs[i]], sems[i]).wait()

    for i in range(_NIX):
        fire(i, i)

    def grp(g, _):
        b = _NIX * g
        for i in range(_NIX):
            drain(i)
            fire(b + i, i)
        return _
    lax.fori_loop(1, nblk // _NIX, grp, None)
    for i in range(_NIX):
        drain(i)

    plsc.subcore_barrier()

    @pl.when(cid == 0)
    def _():
        pltpu.sync_copy(accum.at[pl.ds(sid * sps, sps)],
                        out0_hbm.at[pl.ds(sid * sps, sps)])

    @pl.when(cid == 1)
    def _():
        pltpu.sync_copy(accum.at[pl.ds(sid * sps, sps)],
                        out1_hbm.at[pl.ds(sid * sps, sps)])


def _count_edges(dst, et):
    epw = E // (NC * NS)
    k = pl.kernel(
        _cnt_body,
        out_type=(jax.ShapeDtypeStruct((N * R,), jnp.float32),
                  jax.ShapeDtypeStruct((N * R,), jnp.float32)),
        mesh=_MESH,
        compiler_params=_SC_PARAMS,
        scratch_types=(
            [pltpu.VMEM((epw,), jnp.int32)] * 2
            + [pltpu.VMEM((K,), jnp.int32)] * _NIX
            + [pltpu.VMEM((K,), jnp.float32),
               pltpu.VMEM(((N * R) // NS,), jnp.float32),
               pltpu.VMEM_SHARED((N * R,), jnp.float32)]
            + [pltpu.SemaphoreType.DMA] * _NIX
        ),
    )
    return k(dst, et)


def _inv_kernel(c0_ref, c1_ref, o_ref):
    o_ref[...] = 1.0 / jnp.maximum(c0_ref[...] + c1_ref[...], 1.0)


def _inv_flat(cnt0, cnt1):
    shp = jax.ShapeDtypeStruct((N * R // 128, 128), jnp.float32)
    return pl.pallas_call(_inv_kernel, out_shape=shp)(
        cnt0.reshape(N * R // 128, 128), cnt1.reshape(N * R // 128, 128))


EPS = E // NS
NBLK = EPS // K


def _agg_body(y_hbm, src_hbm, dst_hbm, et_hbm, inv_hbm, out_hbm,
              sv0, sv1, dv0, dv1, ev0, ev1, gx0, gx1, ix0, ix1, ds0, ds1,
              iscl0, iscl1, rows0, rows1, accum,
              semx0, semx1, semr0, semr1, semi0, semi1, sems0, sems1):
    cid = lax.axis_index("c")
    sid = lax.axis_index("s")
    bufs = ((sv0, dv0, ev0, gx0, ix0, ds0, iscl0, rows0,
             semx0, semr0, semi0, sems0),
            (sv1, dv1, ev1, gx1, ix1, ds1, iscl1, rows1,
             semx1, semr1, semi1, sems1))
    base = sid * EPS
    _zero_accum(rows0, accum, sid)
    plsc.subcore_barrier()

    def idx_copies(b, buf):
        svb, dvb, evb = buf[0], buf[1], buf[2]
        semx = buf[8]
        off = base + b * K
        return (pltpu.make_async_copy(src_hbm.at[pl.ds(off, K)], svb, semx),
                pltpu.make_async_copy(dst_hbm.at[pl.ds(off, K)], dvb, semx),
                pltpu.make_async_copy(et_hbm.at[pl.ds(off, K)], evb, semx))

    def fire_idx(b, buf):
        for cp in idx_copies(b, buf):
            cp.start()

    def scat_copy(buf):
        dsb, rowsb, sems = buf[5], buf[7], buf[11]
        return pltpu.make_async_copy(rowsb, accum.at[dsb], sems)

    def fire_gather(b, buf):
        svb, dvb, evb, gxb, ixb, dsb, isclb, rowsb = buf[:8]
        semr, semi = buf[9], buf[10]
        for cp in idx_copies(b, buf):
            cp.wait()

        @pl.when(b >= 2)
        def _():
            scat_copy(buf).wait()
        for z in range(K // LN):
            sl = pl.ds(z * LN, LN)
            s16 = svb[sl]
            e16 = evb[sl]
            d16 = dvb[sl]
            gxb[sl] = (e16 * 2 + cid) * N + s16
            ixb[sl] = d16 * R + e16
            dsb[sl] = d16
        pltpu.make_async_copy(y_hbm.at[gxb], rowsb, semr).start()
        pltpu.make_async_copy(inv_hbm.at[ixb], isclb, semi).start()

    def finish(b, buf):
        gxb, ixb, dsb, isclb, rowsb = buf[3:8]
        semr, semi = buf[9], buf[10]
        pltpu.make_async_copy(y_hbm.at[gxb], rowsb, semr).wait()
        pltpu.make_async_copy(inv_hbm.at[ixb], isclb, semi).wait()

        def scale(g, _):
            eb = g * LN
            for eo in range(LN):
                e = eb + eo
                sp = plsc.load_gather(isclb, [jnp.broadcast_to(e, (LN,))])
                for j in range(8):
                    sl = pl.ds(j * LN, LN)
                    rowsb[e, sl] = rowsb[e, sl] * sp
            return _
        lax.fori_loop(0, K // LN, scale, None)
        scat_copy(buf).start()

    fire_idx(0, bufs[0])
    fire_idx(1, bufs[1])
    fire_gather(0, bufs[0])

    def pair(g, _):
        b = 2 * g
        for i in range(2):
            bi = b + i
            nxt = bufs[(i + 1) % 2]

            @pl.when(bi + 1 < NBLK)
            def _():
                fire_gather(bi + 1, nxt)
            finish(bi, bufs[i])

            @pl.when(bi + 2 < NBLK)
            def _():
                fire_idx(bi + 2, bufs[i])
        return _
    lax.fori_loop(0, NBLK // 2, pair, None)

    for i in range(2):
        scat_copy(bufs[i]).wait()
    plsc.subcore_barrier()
    pltpu.sync_copy(accum.at[pl.ds(sid * NPS, NPS)],
                    out_hbm.at[cid, pl.ds(sid * NPS, NPS), :])


def _aggregate(y2d, src, dst, et, invf):
    k = pl.kernel(
        _agg_body,
        out_type=jax.ShapeDtypeStruct((NC, N, 128), jnp.float32),
        mesh=_MESH,
        compiler_params=_SC_PARAMS,
        scratch_types=(
            [pltpu.VMEM((K,), jnp.int32)] * 12
            + [pltpu.VMEM((K,), jnp.float32)] * 2
            + [pltpu.VMEM((K, 128), jnp.float32)] * 2
            + [pltpu.VMEM_SHARED((N, 128), jnp.float32)]
            + [pltpu.SemaphoreType.DMA] * 8
        ),
    )
    return k(y2d, src, dst, et, invf)


def _mm_kernel(x_ref, w_ref, o_ref):
    r = jnp.dot(x_ref[...], w_ref[...], preferred_element_type=jnp.float32)
    for c in range(o_ref.shape[0]):
        o_ref[c] = r[:, c * 128:(c + 1) * 128]


def _matmul(x, w):
    n, d = x.shape
    m = w.shape[1]
    g = m // 128
    BN = 1000
    return pl.pallas_call(
        _mm_kernel,
        grid=(n // BN,),
        in_specs=[pl.BlockSpec((BN, d), lambda i: (i, 0)),
                  pl.BlockSpec((d, m), lambda i: (0, 0))],
        out_specs=pl.BlockSpec((g, BN, 128), lambda i: (0, i, 0)),
        out_shape=jax.ShapeDtypeStruct((g, n, 128), jnp.float32),
    )(x, w)


def _fin_kernel(a_ref, x_ref, w_ref, b_ref, o_ref, *, res):
    a = jnp.concatenate([a_ref[0], a_ref[1]], axis=-1)
    h = jax.nn.relu(a + jnp.dot(x_ref[...], w_ref[...],
                                preferred_element_type=jnp.float32) + b_ref[...])
    if res:
        h = h + x_ref[...]
    o_ref[...] = h


def _finish(a2, x, w_self, bias, res):
    d = x.shape[1]
    BN = 1000
    return pl.pallas_call(
        functools.partial(_fin_kernel, res=res),
        grid=(N // BN,),
        in_specs=[pl.BlockSpec((NC, BN, 128), lambda i: (0, i, 0)),
                  pl.BlockSpec((BN, d), lambda i: (i, 0)),
                  pl.BlockSpec((d, 256), lambda i: (0, 0)),
                  pl.BlockSpec((1, 256), lambda i: (0, 0))],
        out_specs=pl.BlockSpec((BN, 256), lambda i: (i, 0)),
        out_shape=jax.ShapeDtypeStruct((N, 256), jnp.float32),
    )(a2, x, w_self, bias)


def _gath_body(emb_hbm, ctab_hbm, i1_hbm, i2_hbm, i3_hbm,
               d1a_hbm, d1b_hbm, d2a_hbm, d2b_hbm, cc_hbm,
               iv, gx, r1, sem1):
    cid = lax.axis_index("c")
    sid = lax.axis_index("s")
    wid = cid * NS + sid
    q = B // (NC * NS)
    base = wid * q

    def emb_gather(idx_hbm, half, out_hbm):
        pltpu.sync_copy(idx_hbm.at[pl.ds(base, q)], iv)
        for z in range(q // LN):
            sl = pl.ds(z * LN, LN)
            gx[sl] = iv[sl] * 2 + half
        pltpu.async_copy(emb_hbm.at[gx], r1, sem1).wait()
        pltpu.sync_copy(r1, out_hbm.at[pl.ds(base, q)])

    emb_gather(i1_hbm, 0, d1a_hbm)
    emb_gather(i1_hbm, 1, d1b_hbm)
    emb_gather(i2_hbm, 0, d2a_hbm)
    emb_gather(i2_hbm, 1, d2b_hbm)
    pltpu.sync_copy(i3_hbm.at[pl.ds(base, q)], iv)
    pltpu.async_copy(ctab_hbm.at[iv], r1, sem1).wait()
    pltpu.sync_copy(r1, cc_hbm.at[pl.ds(base, q)])


def _head_gather(emb2, ctab, i1, i2, i3):
    q = B // (NC * NS)
    out = jax.ShapeDtypeStruct((B, 128), jnp.float32)
    k = pl.kernel(
        _gath_body,
        out_type=(out,) * 5,
        mesh=_MESH,
        compiler_params=_SC_PARAMS,
        scratch_types=[
            pltpu.VMEM((q,), jnp.int32),
            pltpu.VMEM((q,), jnp.int32),
            pltpu.VMEM((q, 128), jnp.float32),
            pltpu.SemaphoreType.DMA,
        ],
    )
    return k(emb2, ctab, i1, i2, i3)


def _head_kernel(d1a_ref, d1b_ref, d2a_ref, d2b_ref, cc_ref,
                 wa1_ref, wa2_ref, wb1_ref, wb2_ref, wc_ref, b1_ref,
                 w2_ref, b2_ref, o_ref):
    dot = functools.partial(jnp.dot, preferred_element_type=jnp.float32)
    hid = jax.nn.relu(
        dot(d1a_ref[...], wa1_ref[...]) + dot(d1b_ref[...], wa2_ref[...])
        + dot(d2a_ref[...], wb1_ref[...]) + dot(d2b_ref[...], wb2_ref[...])
        + dot(cc_ref[...], wc_ref[...]) + b1_ref[...])
    o_ref[...] = jnp.sum(hid * w2_ref[...], axis=1, keepdims=True) + b2_ref[...]


def _head(d1a, d1b, d2a, d2b, cc, wm1, bm1, wm2, bm2):
    wc = jnp.pad(wm1[512:], ((0, 64), (0, 0)))
    return pl.pallas_call(
        _head_kernel,
        out_shape=jax.ShapeDtypeStruct((B, 1), jnp.float32),
    )(d1a, d1b, d2a, d2b, cc, wm1[:128], wm1[128:256], wm1[256:384],
      wm1[384:512], wc, bm1.reshape(1, 512), wm2.reshape(1, 512),
      bm2.reshape(1, 1))


def kernel(inputs, node_feature, edge_index, edge_type, W_rel1, W_self1, br1,
           W_rel2, W_self2, br2, ctx_table, Wm1, bm1, Wm2, bm2):
    src = edge_index[0]
    dst = edge_index[1]
    et = edge_type

    cnt0, cnt1 = _count_edges(dst, et)
    invf = _inv_flat(cnt0, cnt1).reshape(N * R)

    Wt1 = W_rel1.reshape(R, 128, 256).transpose(1, 0, 2).reshape(128, R * 256)
    y1 = _matmul(node_feature, Wt1)
    a1 = _aggregate(y1.reshape(N * R * 2, 128), src, dst, et, invf)
    h1 = _finish(a1, node_feature, W_self1, br1.reshape(1, 256), res=False)

    Wt2 = W_rel2.reshape(R, 256, 256).transpose(1, 0, 2).reshape(256, R * 256)
    y2 = _matmul(h1, Wt2)
    a2 = _aggregate(y2.reshape(N * R * 2, 128), src, dst, et, invf)
    h2 = _finish(a2, h1, W_self2, br2.reshape(1, 256), res=True)

    ctab = jnp.pad(ctx_table, ((0, 0), (0, 64)))
    d1a, d1b, d2a, d2b, cc = _head_gather(
        h2.reshape(2 * N, 128), ctab,
        inputs[:, 0], inputs[:, 1], inputs[:, 2])
    out = _head(d1a, d1b, d2a, d2b, cc, Wm1, bm1, Wm2, bm2)
    return out.reshape(B)


# speedup vs baseline: 6.1325x; 1.0020x over previous
import functools

import jax
import jax.numpy as jnp
from jax import lax
from jax.experimental import pallas as pl
from jax.experimental.pallas import tpu as pltpu
from jax.experimental.pallas import tpu_sc as plsc

N = 10000
E = 320000
R = 16
B = 4096

NC = 2
NS = 16
LN = 16
NPS = N // NS
K = 80

_MESH = plsc.VectorSubcoreMesh(
    core_axis_name="c", subcore_axis_name="s", num_cores=NC, num_subcores=NS)
_SC_PARAMS = pltpu.CompilerParams(use_tc_tiling_on_sc=False,
                                  needs_layout_passes=False)

_ZERO16 = functools.partial(jnp.zeros, (LN,), jnp.float32)


def _zero_accum(zbuf, accum, sid):
    def zb(i, _):
        for j in range(8):
            zbuf[i, pl.ds(j * LN, LN)] = _ZERO16()
        return _
    lax.fori_loop(0, K, zb, None)
    for t in range(NPS // K):
        pltpu.sync_copy(zbuf, accum.at[pl.ds(sid * NPS + t * K, K)])
    rem = NPS % K
    if rem:
        pltpu.sync_copy(zbuf.at[pl.ds(0, rem)],
                        accum.at[pl.ds(sid * NPS + (NPS // K) * K, rem)])


_NIX = 5


def _cnt_body(dst_hbm, et_hbm, out0_hbm, out1_hbm, dv, ev,
              ix0, ix1, ix2, ix3, ix4, ones, zbuf, accum,
              sem0, sem1, sem2, sem3, sem4):
    cid = lax.axis_index("c")
    sid = lax.axis_index("s")
    sps = (N * R) // NS
    ixs = (ix0, ix1, ix2, ix3, ix4)
    sems = (sem0, sem1, sem2, sem3, sem4)

    def zb(i, _):
        zbuf[pl.ds(i * LN, LN)] = _ZERO16()
        return _
    lax.fori_loop(0, sps // LN, zb, None)
    pltpu.sync_copy(zbuf, accum.at[pl.ds(sid * sps, sps)])
    for z in range(K // LN):
        ones[pl.ds(z * LN, LN)] = jnp.ones((LN,), jnp.float32)

    epw = E // (NC * NS)
    wid = cid * NS + sid
    nblk = epw // K
    pltpu.sync_copy(dst_hbm.at[pl.ds(wid * epw, epw)], dv)
    pltpu.sync_copy(et_hbm.at[pl.ds(wid * epw, epw)], ev)
    plsc.subcore_barrier()

    def fire(b, i):
        for z in range(K // LN):
            sl = pl.ds(z * LN, LN)
            dsl = pl.ds(b * K + z * LN, LN)
            ixs[i][sl] = dv[dsl] * R + ev[dsl]
        pltpu.make_async_copy(ones, accum.at[ixs[i]], sems[i]).start(add=True)

    def drain(i):
        pltpu.make_async_copy(ones, accum.at[ixs[i]], sems[i]).wait()

    for i in range(_NIX):
        fire(i, i)

    def grp(g, _):
        b = _NIX * g
        for i in range(_NIX):
            drain(i)
            fire(b + i, i)
        return _
    lax.fori_loop(1, nblk // _NIX, grp, None)
    for i in range(_NIX):
        drain(i)

    plsc.subcore_barrier()

    @pl.when(cid == 0)
    def _():
        pltpu.sync_copy(accum.at[pl.ds(sid * sps, sps)],
                        out0_hbm.at[pl.ds(sid * sps, sps)])

    @pl.when(cid == 1)
    def _():
        pltpu.sync_copy(accum.at[pl.ds(sid * sps, sps)],
                        out1_hbm.at[pl.ds(sid * sps, sps)])


def _count_edges(dst, et):
    epw = E // (NC * NS)
    k = pl.kernel(
        _cnt_body,
        out_type=(jax.ShapeDtypeStruct((N * R,), jnp.float32),
                  jax.ShapeDtypeStruct((N * R,), jnp.float32)),
        mesh=_MESH,
        compiler_params=_SC_PARAMS,
        scratch_types=(
            [pltpu.VMEM((epw,), jnp.int32)] * 2
            + [pltpu.VMEM((K,), jnp.int32)] * _NIX
            + [pltpu.VMEM((K,), jnp.float32),
               pltpu.VMEM(((N * R) // NS,), jnp.float32),
               pltpu.VMEM_SHARED((N * R,), jnp.float32)]
            + [pltpu.SemaphoreType.DMA] * _NIX
        ),
    )
    return k(dst, et)


def _inv_kernel(c0_ref, c1_ref, o_ref):
    o_ref[...] = 1.0 / jnp.maximum(c0_ref[...] + c1_ref[...], 1.0)


def _inv_flat(cnt0, cnt1):
    shp = jax.ShapeDtypeStruct((N * R // 128, 128), jnp.float32)
    return pl.pallas_call(_inv_kernel, out_shape=shp)(
        cnt0.reshape(N * R // 128, 128), cnt1.reshape(N * R // 128, 128))


EPS = E // NS
NBLK = EPS // K


def _agg_body(y_hbm, src_hbm, dst_hbm, et_hbm, inv_hbm, out_hbm,
              sv0, sv1, dv0, dv1, ev0, ev1, gx0, gx1, ix0, ix1, ds0, ds1,
              iscl0, iscl1, rows0, rows1, accum,
              semx0, semx1, semr0, semr1, semi0, semi1, sems0, sems1):
    cid = lax.axis_index("c")
    sid = lax.axis_index("s")
    bufs = ((sv0, dv0, ev0, gx0, ix0, ds0, iscl0, rows0,
             semx0, semr0, semi0, sems0),
            (sv1, dv1, ev1, gx1, ix1, ds1, iscl1, rows1,
             semx1, semr1, semi1, sems1))
    base = sid * EPS
    _zero_accum(rows0, accum, sid)
    plsc.subcore_barrier()

    def idx_copies(b, buf):
        svb, dvb, evb = buf[0], buf[1], buf[2]
        semx = buf[8]
        off = base + b * K
        return (pltpu.make_async_copy(src_hbm.at[pl.ds(off, K)], svb, semx),
                pltpu.make_async_copy(dst_hbm.at[pl.ds(off, K)], dvb, semx),
                pltpu.make_async_copy(et_hbm.at[pl.ds(off, K)], evb, semx))

    def fire_idx(b, buf):
        for cp in idx_copies(b, buf):
            cp.start()

    def scat_copy(buf):
        dsb, rowsb, sems = buf[5], buf[7], buf[11]
        return pltpu.make_async_copy(rowsb, accum.at[dsb], sems)

    def fire_gather(b, buf):
        svb, dvb, evb, gxb, ixb, dsb, isclb, rowsb = buf[:8]
        semr, semi = buf[9], buf[10]
        for cp in idx_copies(b, buf):
            cp.wait()

        @pl.when(b >= 2)
        def _():
            scat_copy(buf).wait()
        for z in range(K // LN):
            sl = pl.ds(z * LN, LN)
            s16 = svb[sl]
            e16 = evb[sl]
            d16 = dvb[sl]
            gxb[sl] = (e16 * 2 + cid) * N + s16
            ixb[sl] = d16 * R + e16
            dsb[sl] = d16
        pltpu.make_async_copy(y_hbm.at[gxb], rowsb, semr).start()
        pltpu.make_async_copy(inv_hbm.at[ixb], isclb, semi).start()

    def finish(b, buf):
        gxb, ixb, dsb, isclb, rowsb = buf[3:8]
        semr, semi = buf[9], buf[10]
        pltpu.make_async_copy(y_hbm.at[gxb], rowsb, semr).wait()
        pltpu.make_async_copy(inv_hbm.at[ixb], isclb, semi).wait()

        def scale(g, _):
            eb = g * LN
            for eo in range(LN):
                e = eb + eo
                sp = plsc.load_gather(isclb, [jnp.broadcast_to(e, (LN,))])
                for j in range(8):
                    sl = pl.ds(j * LN, LN)
                    rowsb[e, sl] = rowsb[e, sl] * sp
            return _
        lax.fori_loop(0, K // LN, scale, None)
        scat_copy(buf).start(add=True)

    fire_idx(0, bufs[0])
    fire_idx(1, bufs[1])
    fire_gather(0, bufs[0])

    def pair(g, _):
        b = 2 * g
        for i in range(2):
            bi = b + i
            nxt = bufs[(i + 1) % 2]

            @pl.when(bi + 1 < NBLK)
            def _():
                fire_gather(bi + 1, nxt)
            finish(bi, bufs[i])

            @pl.when(bi + 2 < NBLK)
            def _():
                fire_idx(bi + 2, bufs[i])
        return _
    lax.fori_loop(0, NBLK // 2, pair, None)

    for i in range(2):
        scat_copy(bufs[i]).wait()
    plsc.subcore_barrier()
    pltpu.sync_copy(accum.at[pl.ds(sid * NPS, NPS)],
                    out_hbm.at[cid, pl.ds(sid * NPS, NPS), :])


def _aggregate(y2d, src, dst, et, invf):
    k = pl.kernel(
        _agg_body,
        out_type=jax.ShapeDtypeStruct((NC, N, 128), jnp.float32),
        mesh=_MESH,
        compiler_params=_SC_PARAMS,
        scratch_types=(
            [pltpu.VMEM((K,), jnp.int32)] * 12
            + [pltpu.VMEM((K,), jnp.float32)] * 2
            + [pltpu.VMEM((K, 128), jnp.float32)] * 2
            + [pltpu.VMEM_SHARED((N, 128), jnp.float32)]
            + [pltpu.SemaphoreType.DMA] * 8
        ),
    )
    return k(y2d, src, dst, et, invf)


def _mm_kernel(x_ref, w_ref, o_ref):
    r = jnp.dot(x_ref[...], w_ref[...], preferred_element_type=jnp.float32)
    for c in range(o_ref.shape[0]):
        o_ref[c] = r[:, c * 128:(c + 1) * 128]


def _matmul(x, w):
    n, d = x.shape
    m = w.shape[1]
    g = m // 128
    BN = 1000
    return pl.pallas_call(
        _mm_kernel,
        grid=(n // BN,),
        in_specs=[pl.BlockSpec((BN, d), lambda i: (i, 0)),
                  pl.BlockSpec((d, m), lambda i: (0, 0))],
        out_specs=pl.BlockSpec((g, BN, 128), lambda i: (0, i, 0)),
        out_shape=jax.ShapeDtypeStruct((g, n, 128), jnp.float32),
    )(x, w)


def _fin_kernel(a_ref, x_ref, w_ref, b_ref, o_ref, *, res):
    a = jnp.concatenate([a_ref[0], a_ref[1]], axis=-1)
    h = jax.nn.relu(a + jnp.dot(x_ref[...], w_ref[...],
                                preferred_element_type=jnp.float32) + b_ref[...])
    if res:
        h = h + x_ref[...]
    o_ref[...] = h


def _finish(a2, x, w_self, bias, res):
    d = x.shape[1]
    BN = 1000
    return pl.pallas_call(
        functools.partial(_fin_kernel, res=res),
        grid=(N // BN,),
        in_specs=[pl.BlockSpec((NC, BN, 128), lambda i: (0, i, 0)),
                  pl.BlockSpec((BN, d), lambda i: (i, 0)),
                  pl.BlockSpec((d, 256), lambda i: (0, 0)),
                  pl.BlockSpec((1, 256), lambda i: (0, 0))],
        out_specs=pl.BlockSpec((BN, 256), lambda i: (i, 0)),
        out_shape=jax.ShapeDtypeStruct((N, 256), jnp.float32),
    )(a2, x, w_self, bias)


def _gath_body(emb_hbm, ctab_hbm, i1_hbm, i2_hbm, i3_hbm,
               d1a_hbm, d1b_hbm, d2a_hbm, d2b_hbm, cc_hbm,
               iv, gx, r1, sem1):
    cid = lax.axis_index("c")
    sid = lax.axis_index("s")
    wid = cid * NS + sid
    q = B // (NC * NS)
    base = wid * q

    def emb_gather(idx_hbm, half, out_hbm):
        pltpu.sync_copy(idx_hbm.at[pl.ds(base, q)], iv)
        for z in range(q // LN):
            sl = pl.ds(z * LN, LN)
            gx[sl] = iv[sl] * 2 + half
        pltpu.async_copy(emb_hbm.at[gx], r1, sem1).wait()
        pltpu.sync_copy(r1, out_hbm.at[pl.ds(base, q)])

    emb_gather(i1_hbm, 0, d1a_hbm)
    emb_gather(i1_hbm, 1, d1b_hbm)
    emb_gather(i2_hbm, 0, d2a_hbm)
    emb_gather(i2_hbm, 1, d2b_hbm)
    pltpu.sync_copy(i3_hbm.at[pl.ds(base, q)], iv)
    pltpu.async_copy(ctab_hbm.at[iv], r1, sem1).wait()
    pltpu.sync_copy(r1, cc_hbm.at[pl.ds(base, q)])


def _head_gather(emb2, ctab, i1, i2, i3):
    q = B // (NC * NS)
    out = jax.ShapeDtypeStruct((B, 128), jnp.float32)
    k = pl.kernel(
        _gath_body,
        out_type=(out,) * 5,
        mesh=_MESH,
        compiler_params=_SC_PARAMS,
        scratch_types=[
            pltpu.VMEM((q,), jnp.int32),
            pltpu.VMEM((q,), jnp.int32),
            pltpu.VMEM((q, 128), jnp.float32),
            pltpu.SemaphoreType.DMA,
        ],
    )
    return k(emb2, ctab, i1, i2, i3)


def _head_kernel(d1a_ref, d1b_ref, d2a_ref, d2b_ref, cc_ref,
                 wa1_ref, wa2_ref, wb1_ref, wb2_ref, wc_ref, b1_ref,
                 w2_ref, b2_ref, o_ref):
    dot = functools.partial(jnp.dot, preferred_element_type=jnp.float32)
    hid = jax.nn.relu(
        dot(d1a_ref[...], wa1_ref[...]) + dot(d1b_ref[...], wa2_ref[...])
        + dot(d2a_ref[...], wb1_ref[...]) + dot(d2b_ref[...], wb2_ref[...])
        + dot(cc_ref[...], wc_ref[...]) + b1_ref[...])
    o_ref[...] = jnp.sum(hid * w2_ref[...], axis=1, keepdims=True) + b2_ref[...]


def _head(d1a, d1b, d2a, d2b, cc, wm1, bm1, wm2, bm2):
    wc = jnp.pad(wm1[512:], ((0, 64), (0, 0)))
    return pl.pallas_call(
        _head_kernel,
        out_shape=jax.ShapeDtypeStruct((B, 1), jnp.float32),
    )(d1a, d1b, d2a, d2b, cc, wm1[:128], wm1[128:256], wm1[256:384],
      wm1[384:512], wc, bm1.reshape(1, 512), wm2.reshape(1, 512),
      bm2.reshape(1, 1))


def kernel(inputs, node_feature, edge_index, edge_type, W_rel1, W_self1, br1,
           W_rel2, W_self2, br2, ctx_table, Wm1, bm1, Wm2, bm2):
    src = edge_index[0]
    dst = edge_index[1]
    et = edge_type

    cnt0, cnt1 = _count_edges(dst, et)
    invf = _inv_flat(cnt0, cnt1).reshape(N * R)

    Wt1 = W_rel1.reshape(R, 128, 256).transpose(1, 0, 2).reshape(128, R * 256)
    y1 = _matmul(node_feature, Wt1)
    a1 = _aggregate(y1.reshape(N * R * 2, 128), src, dst, et, invf)
    h1 = _finish(a1, node_feature, W_self1, br1.reshape(1, 256), res=False)

    Wt2 = W_rel2.reshape(R, 256, 256).transpose(1, 0, 2).reshape(256, R * 256)
    y2 = _matmul(h1, Wt2)
    a2 = _aggregate(y2.reshape(N * R * 2, 128), src, dst, et, invf)
    h2 = _finish(a2, h1, W_self2, br2.reshape(1, 256), res=True)

    ctab = jnp.pad(ctx_table, ((0, 0), (0, 64)))
    d1a, d1b, d2a, d2b, cc = _head_gather(
        h2.reshape(2 * N, 128), ctab,
        inputs[:, 0], inputs[:, 1], inputs[:, 2])
    out = _head(d1a, d1b, d2a, d2b, cc, Wm1, bm1, Wm2, bm2)
    return out.reshape(B)
